# baseline (speedup 1.0000x reference)
import functools

import jax
import jax.numpy as jnp
from jax import lax
from jax.experimental import pallas as pl
from jax.experimental.pallas import tpu as pltpu
from jax.experimental.pallas import tpu_sc as plsc

NC, NS = 2, 16
NW = NC * NS
LANES = 16


def _sc_embed(table, idx_t, pos):
    S, B = idx_t.shape
    D = table.shape[1]
    BT = B // NW
    DT = D // 8
    BP = BT + 1
    NG = 4
    mesh = plsc.VectorSubcoreMesh(core_axis_name="c", subcore_axis_name="s")

    @functools.partial(
        pl.kernel,
        out_type=jax.ShapeDtypeStruct((S, DT, NW, 8, BT), jnp.float32),
        mesh=mesh,
        scratch_types=[
            pltpu.VMEM((S, BT), jnp.int32),
            pltpu.VMEM((S, D), jnp.float32),
            pltpu.VMEM((NG, BT, D), jnp.float32),
            pltpu.VMEM((2, D, BP), jnp.float32),
            pltpu.SemaphoreType.DMA,
            pltpu.SemaphoreType.DMA,
            pltpu.SemaphoreType.DMA,
            pltpu.SemaphoreType.DMA,
            pltpu.SemaphoreType.DMA,
            pltpu.SemaphoreType.DMA,
        ],
        compiler_params=pltpu.CompilerParams(
            use_tc_tiling_on_sc=False, needs_layout_passes=False),
    )
    def k(table_hbm, idx_hbm, pos_hbm, out_hbm, idx_v, pos_v, gbufs, obufs,
          gsem0, gsem1, gsem2, gsem3, wsem0, wsem1):
        wid = lax.axis_index("s") * NC + lax.axis_index("c")
        pltpu.sync_copy(idx_hbm.at[:, pl.ds(wid * BT, BT)], idx_v)
        pltpu.sync_copy(pos_hbm, pos_v)
        gsems = (gsem0, gsem1, gsem2, gsem3)
        wsems = (wsem0, wsem1)
        lane = lax.iota(jnp.int32, LANES)

        def fire_gather(s, p):
            pltpu.async_copy(table_hbm.at[idx_v.at[s]], gbufs.at[p], gsems[p])

        def wait_gather(s, p):
            pltpu.make_async_copy(
                table_hbm.at[idx_v.at[s]], gbufs.at[p], gsems[p]).wait()

        def fire_writes(s, p):
            ob = obufs.at[p]
            for dt in range(DT):
                pltpu.async_copy(ob.at[pl.ds(dt * 8, 8), pl.ds(0, BT)],
                                 out_hbm.at[s, dt, wid], wsems[p])

        def drain_writes(s, p):
            ob = obufs.at[p]
            for dt in range(DT):
                pltpu.make_async_copy(ob.at[pl.ds(dt * 8, 8), pl.ds(0, BT)],
                                      out_hbm.at[s, dt, wid], wsems[p]).wait()

        def process(s, pg, po, guard_first):
            gb = gbufs.at[pg]
            ob = obufs.at[po]
            wait_gather(s, pg)
            if guard_first:
                @pl.when(s >= 2)
                def _():
                    drain_writes(s - 2, po)
            else:
                drain_writes(s - 2, po)

            pos_js = [pos_v[s, pl.ds(j * LANES, LANES)]
                      for j in range(D // LANES)]
            row_js = [lane + j * LANES for j in range(D // LANES)]

            @plsc.parallel_loop(0, BT, 1, unroll=8)
            def add_body(b):
                col = lane * 0 + b
                for j in range(D // LANES):
                    v = gb[b, pl.ds(j * LANES, LANES)] + pos_js[j]
                    plsc.store_scatter(ob, [row_js[j], col], v)

            fire_writes(s, po)

        for p in range(2):
            fire_gather(p, p)

        def step_body(s_, carry):
            for u in range(NG):
                s = NG * s_ + u

                @pl.when(s + 2 < S)
                def _():
                    fire_gather(s + 2, (u + 2) % NG)

                process(s, u, u % 2, guard_first=(u < 2))
            return carry

        lax.fori_loop(0, (S - 2) // NG, step_body, 0)
        for s in range(S - 2, S):
            process(s, s % NG, s % 2, guard_first=False)
        for s in range(S - 2, S):
            drain_writes(s, s % 2)

    return k(table, idx_t, pos)


def kernel(inputs, id_table, pos_table):
    B, S = inputs.shape
    V, D = id_table.shape
    idx_t = inputs.T.astype(jnp.int32)
    out = _sc_embed(id_table, idx_t, pos_table)
    return out.transpose(2, 4, 0, 1, 3).reshape(B, S, D)


# speedup vs baseline: 1.0970x; 1.0970x over previous
import functools

import jax
import jax.numpy as jnp
from jax import lax
from jax.experimental import pallas as pl
from jax.experimental.pallas import tpu as pltpu
from jax.experimental.pallas import tpu_sc as plsc

NC, NS = 2, 16
NW = NC * NS
LANES = 16


def _sc_embed(table, idx_t, pos):
    S, B = idx_t.shape
    D = table.shape[1]
    BT = B // NW
    DT = D // 8
    BP = BT + 1
    NG = 4
    mesh = plsc.VectorSubcoreMesh(core_axis_name="c", subcore_axis_name="s")

    @functools.partial(
        pl.kernel,
        out_type=jax.ShapeDtypeStruct((S, DT, NW, 8, BT), jnp.float32),
        mesh=mesh,
        scratch_types=[
            pltpu.VMEM((S, BT), jnp.int32),
            pltpu.VMEM((S, D), jnp.float32),
            pltpu.VMEM((NG, BT, D), jnp.float32),
            pltpu.VMEM((2, D, BP), jnp.float32),
            pltpu.SemaphoreType.DMA,
            pltpu.SemaphoreType.DMA,
            pltpu.SemaphoreType.DMA,
            pltpu.SemaphoreType.DMA,
            pltpu.SemaphoreType.DMA,
            pltpu.SemaphoreType.DMA,
        ],
        compiler_params=pltpu.CompilerParams(
            use_tc_tiling_on_sc=False, needs_layout_passes=False),
    )
    def k(table_hbm, idx_hbm, pos_hbm, out_hbm, idx_v, pos_v, gbufs, obufs,
          gsem0, gsem1, gsem2, gsem3, wsem0, wsem1):
        wid = lax.axis_index("s") * NC + lax.axis_index("c")
        pltpu.sync_copy(idx_hbm.at[:, pl.ds(wid * BT, BT)], idx_v)
        pltpu.sync_copy(pos_hbm, pos_v)
        gsems = (gsem0, gsem1, gsem2, gsem3)
        wsems = (wsem0, wsem1)
        lane = lax.iota(jnp.int32, LANES)

        def fire_gather(s, p):
            pltpu.async_copy(table_hbm.at[idx_v.at[s]], gbufs.at[p], gsems[p])

        def wait_gather(s, p):
            pltpu.make_async_copy(
                table_hbm.at[idx_v.at[s]], gbufs.at[p], gsems[p]).wait()

        def fire_writes(s, p):
            ob = obufs.at[p]
            for dt in range(DT):
                pltpu.async_copy(ob.at[pl.ds(dt * 8, 8), pl.ds(0, BT)],
                                 out_hbm.at[s, dt, wid], wsems[p])

        def drain_writes(s, p):
            ob = obufs.at[p]
            for dt in range(DT):
                pltpu.make_async_copy(ob.at[pl.ds(dt * 8, 8), pl.ds(0, BT)],
                                      out_hbm.at[s, dt, wid], wsems[p]).wait()

        def process(s, pg, po, guard_first):
            gb = gbufs.at[pg]
            ob = obufs.at[po]
            wait_gather(s, pg)
            if guard_first:
                @pl.when(s >= 2)
                def _():
                    drain_writes(s - 2, po)
            else:
                drain_writes(s - 2, po)

            pos_js = [pos_v[s, pl.ds(j * LANES, LANES)]
                      for j in range(D // LANES)]
            row_js = [lane + j * LANES for j in range(D // LANES)]

            @plsc.parallel_loop(0, BT, 1, unroll=4)
            def add_body(b):
                col = lane * 0 + b
                for j in range(D // LANES):
                    v = gb[b, pl.ds(j * LANES, LANES)] + pos_js[j]
                    plsc.store_scatter(ob, [row_js[j], col], v)

            fire_writes(s, po)

        for p in range(2):
            fire_gather(p, p)

        def step_body(s_, carry):
            for u in range(NG):
                s = NG * s_ + u

                @pl.when(s + 2 < S)
                def _():
                    fire_gather(s + 2, (u + 2) % NG)

                process(s, u, u % 2, guard_first=(u < 2))
            return carry

        lax.fori_loop(0, (S - 2) // NG, step_body, 0)
        for s in range(S - 2, S):
            process(s, s % NG, s % 2, guard_first=False)
        for s in range(S - 2, S):
            drain_writes(s, s % 2)

    return k(table, idx_t, pos)


def kernel(inputs, id_table, pos_table):
    B, S = inputs.shape
    V, D = id_table.shape
    idx_t = inputs.T.astype(jnp.int32)
    out = _sc_embed(id_table, idx_t, pos_table)
    return out.transpose(2, 4, 0, 1, 3).reshape(B, S, D)
